# baseline (speedup 1.0000x reference)
import dataclasses
import functools

import jax
import jax.numpy as jnp
from jax import lax
from jax.experimental import pallas as pl
from jax.experimental.pallas import tpu as pltpu
from jax.experimental.pallas import tpu_sc as plsc

N = 10000
E = 320000
D = 128
EPS = 1e-5

NC = 2
NS = 16
NT = NC * NS
EPT = E // NT
CH = 80
NCHUNK = EPT // CH
NBLK = N // CH
BPT = -(-NBLK // NS)

HD = D // NC
CH2 = 128
NBUF = 4
NOUTER = 40
EPADT = NOUTER * NBUF * CH2
EPAD = NS * EPADT

_mesh = plsc.VectorSubcoreMesh(core_axis_name="c", subcore_axis_name="s")

_cp = pltpu.CompilerParams()
for _f, _v in (("needs_layout_passes", False), ("use_tc_tiling_on_sc", False)):
    if _f in pltpu.CompilerParams.__dataclass_fields__:
        _cp = dataclasses.replace(_cp, **{_f: _v})


_tc_cp = pltpu.CompilerParams(vmem_limit_bytes=128 * 1024 * 1024) \
    if "vmem_limit_bytes" in pltpu.CompilerParams.__dataclass_fields__ \
    else pltpu.CompilerParams()


def _bcast_lane(v16, i):
    idx = jnp.full((16, 1), i, jnp.int32)
    dnums = lax.GatherDimensionNumbers(
        offset_dims=(), collapsed_slice_dims=(0,), start_index_map=(0,))
    return lax.gather(v16, idx, dnums, slice_sizes=(1,),
                      mode=lax.GatherScatterMode.PROMISE_IN_BOUNDS)


@functools.partial(
    pl.kernel,
    out_type=jax.ShapeDtypeStruct((NC, N, 16), jnp.float32),
    mesh=_mesh,
    compiler_params=_cp,
    scratch_types=[
        pltpu.VMEM((NCHUNK, CH), jnp.int32),
        pltpu.VMEM((EPT,), jnp.float32),
        pltpu.VMEM((CH, 16), jnp.float32),
        pltpu.VMEM_SHARED((N, 16), jnp.float32),
        pltpu.SemaphoreType.DMA,
    ],
)
def _sc_deg(dst_hbm, w_hbm, out_hbm, idx_v, w_v, buf_v, acc_sh, sem):
    c = lax.axis_index("c")
    s = lax.axis_index("s")
    tid = c * NS + s
    pltpu.sync_copy(dst_hbm.at[tid], idx_v)
    pltpu.sync_copy(w_hbm.at[tid], w_v)

    zf16 = jnp.zeros((16,), jnp.float32)
    zi16 = jnp.zeros((16,), jnp.int32)
    iota16 = lax.iota(jnp.int32, 16)

    @pl.loop(0, CH)
    def _zero_buf(i):
        buf_v[i, :] = zf16

    @pl.loop(0, BPT)
    def _zero_acc(j):
        blk = s + j * NS

        @pl.when(blk < NBLK)
        def _():
            pltpu.sync_copy(buf_v, acc_sh.at[pl.ds(blk * CH, CH)])
    plsc.subcore_barrier()

    @pl.loop(0, NCHUNK)
    def _chunk(g):
        for k in range(CH // 16):
            off = pl.multiple_of(g * CH + k * 16, 16)
            w16 = w_v[pl.ds(off, 16)]
            plsc.store_scatter(buf_v, [iota16 + (k * 16), zi16], w16)
        pltpu.sync_copy(buf_v, acc_sh.at[idx_v.at[g]], add=True)

    plsc.subcore_barrier()

    @pl.loop(0, BPT)
    def _writeback(j):
        blk = s + j * NS

        @pl.when(blk < NBLK)
        def _():
            pltpu.sync_copy(acc_sh.at[pl.ds(blk * CH, CH)],
                            out_hbm.at[c].at[pl.ds(blk * CH, CH)])


@functools.partial(
    pl.kernel,
    out_type=jax.ShapeDtypeStruct((NC, N, HD), jnp.float32),
    mesh=_mesh,
    compiler_params=_cp,
    scratch_types=[
        pltpu.VMEM((2, 3, NBUF, CH2), jnp.int32),
        pltpu.VMEM((NBUF, CH2, HD), jnp.float32),
        pltpu.VMEM((NBUF, CH2, HD), jnp.float32),
        pltpu.VMEM_SHARED((N, HD), jnp.float32),
    ] + [pltpu.SemaphoreType.DMA] * (2 * NBUF + 1),
)
def _sc_agg(h_hbm, ed_hbm, out_hbm, e_v, rows_v, sc_v, acc_sh, *sems):
    sem_g = sems[:NBUF]
    sem_s = sems[NBUF:2 * NBUF]
    sem_e = sems[2 * NBUF]
    c = lax.axis_index("c")
    s = lax.axis_index("s")

    zf16 = jnp.zeros((16,), jnp.float32)

    @pl.loop(0, CH2)
    def _zero_rows(i):
        for cg in range(HD // 16):
            sc_v[0, i, pl.ds(cg * 16, 16)] = zf16

    @pl.loop(0, BPT)
    def _zero_acc(j):
        blk = s + j * NS

        @pl.when(blk < NBLK)
        def _():
            pltpu.sync_copy(sc_v.at[0].at[pl.ds(0, CH)],
                            acc_sh.at[pl.ds(blk * CH, CH)])
    plsc.subcore_barrier()

    def _gather_fire(sl, b):
        pltpu.async_copy(h_hbm.at[c].at[e_v.at[sl].at[0].at[b]],
                         rows_v.at[b], sem_g[b])

    def _gather_wait(sl, b):
        pltpu.make_async_copy(h_hbm.at[c].at[e_v.at[sl].at[0].at[b]],
                              rows_v.at[b], sem_g[b]).wait()

    def _scatter_fire(sl, b):
        pltpu.async_copy(sc_v.at[b], acc_sh.at[e_v.at[sl].at[1].at[b]],
                         sem_s[b], add=True)

    def _scatter_wait(sl, b):
        pltpu.make_async_copy(sc_v.at[b], acc_sh.at[e_v.at[sl].at[1].at[b]],
                              sem_s[b]).wait()

    def _scale(sl, b):
        @pl.loop(0, CH2 // 16)
        def _scale_k(k):
            w16 = plsc.bitcast(e_v[sl, 2, b, pl.ds(k * 16, 16)], jnp.float32)
            for i in range(16):
                wb = _bcast_lane(w16, i)
                e = k * 16 + i
                for cg in range(HD // 16):
                    sl2 = pl.ds(cg * 16, 16)
                    sc_v[b, e, sl2] = rows_v[b, e, sl2] * wb

    pltpu.sync_copy(ed_hbm.at[s].at[0], e_v.at[0])
    for b in range(NBUF):
        _gather_fire(0, b)

    @pl.loop(0, NOUTER)
    def _outer(t):
        slot = lax.rem(t, 2)
        nslot = 1 - slot

        @pl.when(t > 0)
        def _():
            for b in range(NBUF):
                _scatter_wait(nslot, b)

        @pl.when(t + 1 < NOUTER)
        def _():
            pltpu.async_copy(ed_hbm.at[s].at[t + 1], e_v.at[nslot], sem_e)

        for b in range(NBUF):
            _gather_wait(slot, b)
            _scale(slot, b)
            _scatter_fire(slot, b)

        @pl.when(t + 1 < NOUTER)
        def _():
            pltpu.make_async_copy(ed_hbm.at[s].at[t + 1], e_v.at[nslot],
                                  sem_e).wait()
            for b in range(NBUF):
                _gather_fire(nslot, b)

    for b in range(NBUF):
        _scatter_wait((NOUTER - 1) % 2, b)
    plsc.subcore_barrier()

    @pl.loop(0, BPT)
    def _writeback(j):
        blk = s + j * NS

        @pl.when(blk < NBLK)
        def _():
            pltpu.sync_copy(acc_sh.at[pl.ds(blk * CH, CH)],
                            out_hbm.at[c].at[pl.ds(blk * CH, CH)])


def _tc_pre_body(x_ref, w1_ref, degp_ref, h1_ref, h1s_ref, dis_ref):
    deg = degp_ref[0, :, 0:1] + degp_ref[1, :, 0:1] + 1.0
    dis = jnp.where(deg > 0,
                    lax.rsqrt(jnp.maximum(deg, 1e-12)),
                    jnp.zeros_like(deg))
    dis_ref[...] = dis
    h1 = jnp.dot(x_ref[...], w1_ref[...],
                 preferred_element_type=jnp.float32,
                 precision=lax.Precision.HIGHEST)
    h1_ref[...] = h1
    hs = h1 * dis
    h1s_ref[0] = hs[:, 0:HD]
    h1s_ref[1] = hs[:, HD:D]


def _tc_pre(x, W1, degp):
    return pl.pallas_call(
        _tc_pre_body,
        compiler_params=_tc_cp,
        out_shape=(jax.ShapeDtypeStruct((N, D), jnp.float32),
                   jax.ShapeDtypeStruct((NC, N, HD), jnp.float32),
                   jax.ShapeDtypeStruct((N, 1), jnp.float32)),
    )(x, W1, degp)


def _tc_mid_body(accp_ref, dis_ref, h1_ref, b1_ref, g_ref, be_ref, w2_ref,
                 h2_ref, h2s_ref):
    dis = dis_ref[...]
    acc = jnp.concatenate([accp_ref[0], accp_ref[1]], axis=1)
    h = dis * acc + (dis * dis) * h1_ref[...] + b1_ref[...]
    mu = jnp.mean(h)
    hc = h - mu
    std = jnp.sqrt(jnp.mean(hc * hc))
    y = hc / (std + EPS) * g_ref[...] + be_ref[...]
    r = jnp.maximum(y, 0.0)
    h2 = jnp.dot(r, w2_ref[...],
                 preferred_element_type=jnp.float32,
                 precision=lax.Precision.HIGHEST)
    h2_ref[...] = h2
    hs = h2 * dis
    h2s_ref[0] = hs[:, 0:HD]
    h2s_ref[1] = hs[:, HD:D]


def _tc_mid(accp, dis, h1, b1, gamma, beta, W2):
    return pl.pallas_call(
        _tc_mid_body,
        compiler_params=_tc_cp,
        out_shape=(jax.ShapeDtypeStruct((N, D), jnp.float32),
                   jax.ShapeDtypeStruct((NC, N, HD), jnp.float32)),
    )(accp, dis, h1, b1, gamma, beta, W2)


def _tc_post_body(accp_ref, dis_ref, h2_ref, b2_ref, out_ref):
    dis = dis_ref[...]
    acc = jnp.concatenate([accp_ref[0], accp_ref[1]], axis=1)
    out_ref[...] = dis * acc + (dis * dis) * h2_ref[...] + b2_ref[...]


def _tc_post(accp, dis, h2, b2):
    return pl.pallas_call(
        _tc_post_body,
        compiler_params=_tc_cp,
        out_shape=jax.ShapeDtypeStruct((N, D), jnp.float32),
    )(accp, dis, h2, b2)


def kernel(x, edge_index, edge_attr, W1, b1, gamma, beta, W2, b2):
    src = edge_index[0].astype(jnp.int32)
    dst = edge_index[1].astype(jnp.int32)
    w = edge_attr
    dst3 = dst.reshape(NT, NCHUNK, CH)
    w2d = w.reshape(NT, EPT)
    padi = jnp.zeros((EPAD - E,), jnp.int32)
    srcR = jnp.concatenate([src, padi]).reshape(NS, NOUTER, 1, NBUF, CH2)
    dstR = jnp.concatenate([dst, padi]).reshape(NS, NOUTER, 1, NBUF, CH2)
    wR = jnp.concatenate([lax.bitcast_convert_type(w, jnp.int32), padi]
                         ).reshape(NS, NOUTER, 1, NBUF, CH2)
    edata = jnp.concatenate([srcR, dstR, wR], axis=2)
    b1r = b1.reshape(1, D)
    gr = gamma.reshape(1, D)
    ber = beta.reshape(1, D)
    b2r = b2.reshape(1, D)

    degp = _sc_deg(dst3, w2d)
    h1, h1s, dis = _tc_pre(x, W1, degp)
    acc1 = _sc_agg(h1s, edata)
    h2, h2s = _tc_mid(acc1, dis, h1, b1r, gr, ber, W2)
    acc2 = _sc_agg(h2s, edata)
    return _tc_post(acc2, dis, h2, b2r)


# speedup vs baseline: 1.8544x; 1.8544x over previous
import dataclasses
import functools

import jax
import jax.numpy as jnp
from jax import lax
from jax.experimental import pallas as pl
from jax.experimental.pallas import tpu as pltpu
from jax.experimental.pallas import tpu_sc as plsc

N = 10000
E = 320000
D = 128
EPS = 1e-5

NC = 2
NS = 16
NT = NC * NS
EPT = E // NT
CH = 80
NCHUNK = EPT // CH
NBLK = N // CH
BPT = -(-NBLK // NS)

HD = D // NC
CH2 = 80
NBUF = 5
NOUTER = 50
EPADT = NOUTER * NBUF * CH2
EPAD = NS * EPADT

_mesh = plsc.VectorSubcoreMesh(core_axis_name="c", subcore_axis_name="s")

_cp = pltpu.CompilerParams()
for _f, _v in (("needs_layout_passes", False), ("use_tc_tiling_on_sc", False)):
    if _f in pltpu.CompilerParams.__dataclass_fields__:
        _cp = dataclasses.replace(_cp, **{_f: _v})


_tc_cp = pltpu.CompilerParams(vmem_limit_bytes=128 * 1024 * 1024) \
    if "vmem_limit_bytes" in pltpu.CompilerParams.__dataclass_fields__ \
    else pltpu.CompilerParams()


def _bcast_lane(v16, i):
    idx = jnp.full((16, 1), i, jnp.int32)
    dnums = lax.GatherDimensionNumbers(
        offset_dims=(), collapsed_slice_dims=(0,), start_index_map=(0,))
    return lax.gather(v16, idx, dnums, slice_sizes=(1,),
                      mode=lax.GatherScatterMode.PROMISE_IN_BOUNDS)


@functools.partial(
    pl.kernel,
    out_type=jax.ShapeDtypeStruct((NC, N, 16), jnp.float32),
    mesh=_mesh,
    compiler_params=_cp,
    scratch_types=[
        pltpu.VMEM((NCHUNK, CH), jnp.int32),
        pltpu.VMEM((EPT,), jnp.float32),
        pltpu.VMEM((CH, 16), jnp.float32),
        pltpu.VMEM_SHARED((N, 16), jnp.float32),
        pltpu.SemaphoreType.DMA,
    ],
)
def _sc_deg(dst_hbm, w_hbm, out_hbm, idx_v, w_v, buf_v, acc_sh, sem):
    c = lax.axis_index("c")
    s = lax.axis_index("s")
    tid = c * NS + s
    pltpu.sync_copy(dst_hbm.at[tid], idx_v)
    pltpu.sync_copy(w_hbm.at[tid], w_v)

    zf16 = jnp.zeros((16,), jnp.float32)
    zi16 = jnp.zeros((16,), jnp.int32)
    iota16 = lax.iota(jnp.int32, 16)

    @pl.loop(0, CH)
    def _zero_buf(i):
        buf_v[i, :] = zf16

    @pl.loop(0, BPT)
    def _zero_acc(j):
        blk = s + j * NS

        @pl.when(blk < NBLK)
        def _():
            pltpu.sync_copy(buf_v, acc_sh.at[pl.ds(blk * CH, CH)])
    plsc.subcore_barrier()

    @pl.loop(0, NCHUNK)
    def _chunk(g):
        for k in range(CH // 16):
            off = pl.multiple_of(g * CH + k * 16, 16)
            w16 = w_v[pl.ds(off, 16)]
            plsc.store_scatter(buf_v, [iota16 + (k * 16), zi16], w16)
        pltpu.sync_copy(buf_v, acc_sh.at[idx_v.at[g]], add=True)

    plsc.subcore_barrier()

    @pl.loop(0, BPT)
    def _writeback(j):
        blk = s + j * NS

        @pl.when(blk < NBLK)
        def _():
            pltpu.sync_copy(acc_sh.at[pl.ds(blk * CH, CH)],
                            out_hbm.at[c].at[pl.ds(blk * CH, CH)])


@functools.partial(
    pl.kernel,
    out_type=jax.ShapeDtypeStruct((NC, N, HD), jnp.float32),
    mesh=_mesh,
    compiler_params=_cp,
    scratch_types=[
        pltpu.VMEM((2, 3, NBUF, CH2), jnp.int32),
        pltpu.VMEM((NBUF, CH2, HD), jnp.float32),
        pltpu.VMEM((NBUF, CH2, HD), jnp.float32),
        pltpu.VMEM_SHARED((N, HD), jnp.float32),
    ] + [pltpu.SemaphoreType.DMA] * (2 * NBUF + 1),
)
def _sc_agg(h_hbm, ed_hbm, out_hbm, e_v, rows_v, sc_v, acc_sh, *sems):
    sem_g = sems[:NBUF]
    sem_s = sems[NBUF:2 * NBUF]
    sem_e = sems[2 * NBUF]
    c = lax.axis_index("c")
    s = lax.axis_index("s")

    zf16 = jnp.zeros((16,), jnp.float32)

    @pl.loop(0, CH2)
    def _zero_rows(i):
        for cg in range(HD // 16):
            sc_v[0, i, pl.ds(cg * 16, 16)] = zf16

    @pl.loop(0, BPT)
    def _zero_acc(j):
        blk = s + j * NS

        @pl.when(blk < NBLK)
        def _():
            pltpu.sync_copy(sc_v.at[0].at[pl.ds(0, CH)],
                            acc_sh.at[pl.ds(blk * CH, CH)])
    plsc.subcore_barrier()

    def _gather_fire(sl, b):
        pltpu.async_copy(h_hbm.at[c].at[e_v.at[sl].at[0].at[b]],
                         rows_v.at[b], sem_g[b])

    def _gather_wait(sl, b):
        pltpu.make_async_copy(h_hbm.at[c].at[e_v.at[sl].at[0].at[b]],
                              rows_v.at[b], sem_g[b]).wait()

    def _scatter_fire(sl, b):
        pltpu.async_copy(sc_v.at[b], acc_sh.at[e_v.at[sl].at[1].at[b]],
                         sem_s[b], add=True)

    def _scatter_wait(sl, b):
        pltpu.make_async_copy(sc_v.at[b], acc_sh.at[e_v.at[sl].at[1].at[b]],
                              sem_s[b]).wait()

    def _scale(sl, b):
        @pl.loop(0, CH2 // 16)
        def _scale_k(k):
            w16 = plsc.bitcast(e_v[sl, 2, b, pl.ds(k * 16, 16)], jnp.float32)
            for i in range(16):
                wb = _bcast_lane(w16, i)
                e = k * 16 + i
                for cg in range(HD // 16):
                    sl2 = pl.ds(cg * 16, 16)
                    sc_v[b, e, sl2] = rows_v[b, e, sl2] * wb

    pltpu.sync_copy(ed_hbm.at[s].at[0], e_v.at[0])
    for b in range(NBUF):
        _gather_fire(0, b)

    @pl.loop(0, NOUTER)
    def _outer(t):
        slot = lax.rem(t, 2)
        nslot = 1 - slot

        @pl.when(t > 0)
        def _():
            for b in range(NBUF):
                _scatter_wait(nslot, b)

        @pl.when(t + 1 < NOUTER)
        def _():
            pltpu.async_copy(ed_hbm.at[s].at[t + 1], e_v.at[nslot], sem_e)

        for b in range(NBUF):
            _gather_wait(slot, b)
            _scale(slot, b)
            _scatter_fire(slot, b)

        @pl.when(t + 1 < NOUTER)
        def _():
            pltpu.make_async_copy(ed_hbm.at[s].at[t + 1], e_v.at[nslot],
                                  sem_e).wait()
            for b in range(NBUF):
                _gather_fire(nslot, b)

    for b in range(NBUF):
        _scatter_wait((NOUTER - 1) % 2, b)
    plsc.subcore_barrier()

    @pl.loop(0, BPT)
    def _writeback(j):
        blk = s + j * NS

        @pl.when(blk < NBLK)
        def _():
            pltpu.sync_copy(acc_sh.at[pl.ds(blk * CH, CH)],
                            out_hbm.at[c].at[pl.ds(blk * CH, CH)])


def _tc_pre_body(x_ref, w1_ref, degp_ref, h1_ref, h1s_ref, dis_ref):
    deg = degp_ref[0, :, 0:1] + degp_ref[1, :, 0:1] + 1.0
    dis = jnp.where(deg > 0,
                    lax.rsqrt(jnp.maximum(deg, 1e-12)),
                    jnp.zeros_like(deg))
    dis_ref[...] = dis
    h1 = jnp.dot(x_ref[...], w1_ref[...],
                 preferred_element_type=jnp.float32,
                 precision=lax.Precision.HIGHEST)
    h1_ref[...] = h1
    hs = h1 * dis
    h1s_ref[0] = hs[:, 0:HD]
    h1s_ref[1] = hs[:, HD:D]


def _tc_pre(x, W1, degp):
    return pl.pallas_call(
        _tc_pre_body,
        compiler_params=_tc_cp,
        out_shape=(jax.ShapeDtypeStruct((N, D), jnp.float32),
                   jax.ShapeDtypeStruct((NC, N, HD), jnp.float32),
                   jax.ShapeDtypeStruct((N, 1), jnp.float32)),
    )(x, W1, degp)


def _tc_mid_body(accp_ref, dis_ref, h1_ref, b1_ref, g_ref, be_ref, w2_ref,
                 h2_ref, h2s_ref):
    dis = dis_ref[...]
    acc = jnp.concatenate([accp_ref[0], accp_ref[1]], axis=1)
    h = dis * acc + (dis * dis) * h1_ref[...] + b1_ref[...]
    mu = jnp.mean(h)
    hc = h - mu
    std = jnp.sqrt(jnp.mean(hc * hc))
    y = hc / (std + EPS) * g_ref[...] + be_ref[...]
    r = jnp.maximum(y, 0.0)
    h2 = jnp.dot(r, w2_ref[...],
                 preferred_element_type=jnp.float32,
                 precision=lax.Precision.HIGHEST)
    h2_ref[...] = h2
    hs = h2 * dis
    h2s_ref[0] = hs[:, 0:HD]
    h2s_ref[1] = hs[:, HD:D]


def _tc_mid(accp, dis, h1, b1, gamma, beta, W2):
    return pl.pallas_call(
        _tc_mid_body,
        compiler_params=_tc_cp,
        out_shape=(jax.ShapeDtypeStruct((N, D), jnp.float32),
                   jax.ShapeDtypeStruct((NC, N, HD), jnp.float32)),
    )(accp, dis, h1, b1, gamma, beta, W2)


def _tc_post_body(accp_ref, dis_ref, h2_ref, b2_ref, out_ref):
    dis = dis_ref[...]
    acc = jnp.concatenate([accp_ref[0], accp_ref[1]], axis=1)
    out_ref[...] = dis * acc + (dis * dis) * h2_ref[...] + b2_ref[...]


def _tc_post(accp, dis, h2, b2):
    return pl.pallas_call(
        _tc_post_body,
        compiler_params=_tc_cp,
        out_shape=jax.ShapeDtypeStruct((N, D), jnp.float32),
    )(accp, dis, h2, b2)


def kernel(x, edge_index, edge_attr, W1, b1, gamma, beta, W2, b2):
    src = edge_index[0].astype(jnp.int32)
    dst = edge_index[1].astype(jnp.int32)
    w = edge_attr
    dst3 = dst.reshape(NT, NCHUNK, CH)
    w2d = w.reshape(NT, EPT)
    padi = jnp.zeros((EPAD - E,), jnp.int32)
    srcR = jnp.concatenate([src, padi]).reshape(NS, NOUTER, 1, NBUF, CH2)
    dstR = jnp.concatenate([dst, padi]).reshape(NS, NOUTER, 1, NBUF, CH2)
    wR = jnp.concatenate([lax.bitcast_convert_type(w, jnp.int32), padi]
                         ).reshape(NS, NOUTER, 1, NBUF, CH2)
    edata = jnp.concatenate([srcR, dstR, wR], axis=2)
    b1r = b1.reshape(1, D)
    gr = gamma.reshape(1, D)
    ber = beta.reshape(1, D)
    b2r = b2.reshape(1, D)

    degp = _sc_deg(dst3, w2d)
    h1, h1s, dis = _tc_pre(x, W1, degp)
    acc1 = _sc_agg(h1s, edata)
    h2, h2s = _tc_mid(acc1, dis, h1, b1r, gr, ber, W2)
    acc2 = _sc_agg(h2s, edata)
    return _tc_post(acc2, dis, h2, b2r)
